# baseline (speedup 1.0000x reference)
import functools

import jax
import jax.numpy as jnp
from jax import lax
from jax.experimental import pallas as pl
from jax.experimental.pallas import tpu as pltpu
from jax.experimental.pallas import tpu_sc as plsc

_B, _D = 16384, 128
_T = 1000
_NC, _NS, _L = 2, 16, 16
_NW = _NC * _NS
_BPW = _B // _NW


def _gather_coeffs(table1, table2, timesteps):
    mesh = plsc.VectorSubcoreMesh(core_axis_name="c", subcore_axis_name="s")

    @functools.partial(
        pl.kernel,
        out_type=(
            jax.ShapeDtypeStruct((_B,), jnp.float32),
            jax.ShapeDtypeStruct((_B,), jnp.float32),
        ),
        mesh=mesh,
        compiler_params=pltpu.CompilerParams(needs_layout_passes=False, use_tc_tiling_on_sc=True, skip_device_barrier=True),
        scratch_types=[
            pltpu.VMEM((_T,), jnp.float32),
            pltpu.VMEM((_T,), jnp.float32),
            pltpu.VMEM((_BPW,), jnp.int32),
            pltpu.VMEM((_BPW,), jnp.float32),
            pltpu.VMEM((_BPW,), jnp.float32),
            pltpu.SemaphoreType.DMA,
            pltpu.SemaphoreType.DMA,
            pltpu.SemaphoreType.DMA,
        ],
    )
    def gather_kernel(t1_hbm, t2_hbm, ts_hbm, s1_hbm, s2_hbm,
                      t1_v, t2_v, idx_v, s1_v, s2_v, sem1, sem2, sem3):
        wid = lax.axis_index("s") * _NC + lax.axis_index("c")
        base = wid * _BPW
        c1 = pltpu.async_copy(t1_hbm, t1_v, sem1)
        c2 = pltpu.async_copy(t2_hbm, t2_v, sem2)
        c3 = pltpu.async_copy(ts_hbm.at[pl.ds(base, _BPW)], idx_v, sem3)
        c1.wait()
        c2.wait()
        c3.wait()

        for i in range(_BPW // _L):
            sl = pl.ds(i * _L, _L)
            idx = idx_v[sl]
            s1_v[sl] = plsc.load_gather(t1_v, [idx])
            s2_v[sl] = plsc.load_gather(t2_v, [idx])

        o1 = pltpu.async_copy(s1_v, s1_hbm.at[pl.ds(base, _BPW)], sem1)
        o2 = pltpu.async_copy(s2_v, s2_hbm.at[pl.ds(base, _BPW)], sem2)
        o1.wait()
        o2.wait()

    return gather_kernel(table1, table2, timesteps)


def _blend(s1, s2, x_start, x_noise):
    bs = 8192

    def body(s1_ref, s2_ref, xs_ref, xn_ref, o_ref):
        c1 = s1_ref[...].reshape(bs, 1)
        c2 = s2_ref[...].reshape(bs, 1)
        o_ref[...] = c1 * xs_ref[...] + c2 * xn_ref[...]

    return pl.pallas_call(
        body,
        grid=(_B // bs,),
        in_specs=[
            pl.BlockSpec((bs,), lambda i: (i,)),
            pl.BlockSpec((bs,), lambda i: (i,)),
            pl.BlockSpec((bs, _D), lambda i: (i, 0)),
            pl.BlockSpec((bs, _D), lambda i: (i, 0)),
        ],
        out_specs=pl.BlockSpec((bs, _D), lambda i: (i, 0)),
        out_shape=jax.ShapeDtypeStruct((_B, _D), jnp.float32),
    )(s1, s2, x_start, x_noise)


def kernel(x_start, x_noise, timesteps, sqrt_alphas_cumprod,
           sqrt_one_minus_alphas_cumprod):
    s1, s2 = _gather_coeffs(sqrt_alphas_cumprod,
                            sqrt_one_minus_alphas_cumprod, timesteps)
    return _blend(s1, s2, x_start, x_noise)
